# baseline (speedup 1.0000x reference)
import functools

import jax
import jax.numpy as jnp
from jax import lax
from jax.experimental import pallas as pl
from jax.experimental.pallas import tpu as pltpu
from jax.experimental.pallas import tpu_sc as plsc

N = 10000
E = 160000
D = 256
HALF = 128
NG = 64
NT = 16
EPT = E // NT
C = 80
NCHUNK = EPT // C
RCH = 80
NRC = N // RCH
RQ = (NRC + NT - 1) // NT


def _seg_body(with_deg, *refs):
    xa, xp, cw, cc, cr = refs[:5]
    rest = refs[5:]
    if with_deg:
        s_w, s_c, s_r, degs = rest[:4]
        rest = rest[4:]
    else:
        s_w, s_c, s_r = rest[:3]
        degs = None
        rest = rest[3:]
    acc = rest[0]
    cidx = list(rest[1:4])
    sidx = list(rest[4:7])
    didx = list(rest[7:10])
    rows = list(rest[10:13])
    semi, semg, sems = rest[13:16]

    c = lax.axis_index("c")
    t = lax.axis_index("s")
    xoff = c * N

    def fill_rows0(val):
        def _f(k, carry):
            rows[0][k // 8, pl.ds((k % 8) * 16, 16)] = jnp.full(
                (16,), val, jnp.float32)
            return carry
        lax.fori_loop(0, RCH * 8, _f, 0)

    def zero_acc():
        fill_rows0(0.0)
        for q in range(RQ):
            k = q * NT + t
            @pl.when(k < NRC)
            def _():
                pltpu.sync_copy(rows[0], acc.at[pl.ds(k * RCH, RCH)])

    def drain_acc(out_ref, slot, off):
        for q in range(RQ):
            k = q * NT + t
            @pl.when(k < NRC)
            def _():
                pltpu.sync_copy(acc.at[pl.ds(k * RCH, RCH)], rows[0])
                if slot is None:
                    pltpu.sync_copy(rows[0], out_ref.at[pl.ds(off + k * RCH, RCH)])
                else:
                    pltpu.sync_copy(rows[0], out_ref.at[slot, pl.ds(k * RCH, RCH)])

    def run_phase(comb_ref, out_ref, slot, x_ref, jlo=0, n=NCHUNK):
        gather = x_ref is not None
        zero_acc()
        if not gather:
            fill_rows0(1.0)
        plsc.subcore_barrier()
        tbase = t * (2 * EPT)

        def load_cidx(jj, p):
            pltpu.async_copy(
                comb_ref.at[pl.ds(tbase + (jlo + jj) * 2 * C, 2 * C)],
                cidx[p], semi)

        def build(jj, p):
            pltpu.make_async_copy(
                comb_ref.at[pl.ds(tbase + (jlo + jj) * 2 * C, 2 * C)],
                cidx[p], semi).wait()
            for k in range(C // 16):
                if gather:
                    sidx[p][pl.ds(k * 16, 16)] = (
                        cidx[p][pl.ds(k * 16, 16)] + xoff)
                didx[p][pl.ds(k * 16, 16)] = cidx[p][pl.ds(C + k * 16, 16)]

        def gath(p):
            pltpu.async_copy(x_ref.at[sidx[p]], rows[p], semg)

        def wait_gath(p):
            pltpu.make_async_copy(x_ref.at[sidx[p]], rows[p], semg).wait()

        def scat(p):
            src = rows[p] if gather else rows[0]
            pltpu.async_copy(src, acc.at[didx[p]], sems, add=True)

        def wait_scat(p):
            src = rows[p] if gather else rows[0]
            pltpu.make_async_copy(src, acc.at[didx[p]], sems).wait()

        load_cidx(0, 0)
        load_cidx(1, 1)
        build(0, 0)
        if gather:
            gath(0)

        def emit_iter(j, jj):
            p = j % 3
            p1 = (p + 1) % 3
            p2 = (p + 2) % 3
            if gather:
                wait_gath(p)
            scat(p)
            if j >= 2:
                wait_scat(p1)
            if j + 2 < n:
                load_cidx(jj + 2, p2)
            if j + 1 < n:
                build(jj + 1, p1)
                if gather:
                    gath(p1)

        steady = (n - 4) // 3
        tail = 3 * steady + 2
        emit_iter(0, 0)
        emit_iter(1, 1)

        def body(k, carry):
            j = 2 + 3 * k
            emit_iter(2, j)
            emit_iter(3, j + 1)
            emit_iter(4, j + 2)
            return carry
        lax.fori_loop(0, steady, body, 0)
        for j in range(tail, n):
            emit_iter(j, j)
        wait_scat((n - 2) % 3)
        wait_scat((n - 1) % 3)
        plsc.subcore_barrier()
        if slot is None:
            drain_acc(out_ref, None, xoff)
        else:
            drain_acc(out_ref, slot, 0)
        plsc.subcore_barrier()

    run_phase(cw, s_w, None, xa)
    run_phase(cc, s_c, None, xp)
    run_phase(cr, s_r, None, xp)
    if with_deg:
        @pl.when(c == 0)
        def _():
            run_phase(cw, degs, 0, None)
            run_phase(cr, degs, 2, None, 0, NCHUNK // 2)

        @pl.when(c == 1)
        def _():
            run_phase(cc, degs, 1, None)
            run_phase(cr, degs, 3, None, NCHUNK // 2, NCHUNK - NCHUNK // 2)


def _make_seg_kernel(with_deg):
    outs = [jax.ShapeDtypeStruct((2 * N, HALF), jnp.float32) for _ in range(3)]
    if with_deg:
        outs.append(jax.ShapeDtypeStruct((4, N, HALF), jnp.float32))
    scratch = (
        [pltpu.VMEM_SHARED((N, HALF), jnp.float32)]
        + [pltpu.VMEM((2 * C,), jnp.int32)] * 3
        + [pltpu.VMEM((C,), jnp.int32)] * 3
        + [pltpu.VMEM((C,), jnp.int32)] * 3
        + [pltpu.VMEM((C, HALF), jnp.float32)] * 3
        + [pltpu.SemaphoreType.DMA] * 3
    )
    mesh = plsc.VectorSubcoreMesh(core_axis_name="c", subcore_axis_name="s",
                                  num_cores=2, num_subcores=NT)
    return pl.kernel(
        functools.partial(_seg_body, with_deg),
        out_type=tuple(outs),
        mesh=mesh,
        scratch_types=scratch,
    )


_seg_l0 = _make_seg_kernel(True)
_seg_l1 = _make_seg_kernel(False)

R = 1000
GRID = N // R


_CD = (((1,), (1,)), ((), ()))


def _halves(x):
    return x[:, 0:HALF], x[:, HALF:D]


def _paper_body(sw0, sw1, dw, sc0, sc1, dc, xp0, xp1, wlw, wlc, wrw, wrc,
                bw, bc, out):
    rw = 1.0 / jnp.maximum(dw[...], 1.0)
    rc = 1.0 / jnp.maximum(dc[...], 1.0)
    wr = wrw[...] + wrc[...]
    f32 = jnp.float32
    wlw0, wlw1 = _halves(wlw[...])
    wlc0, wlc1 = _halves(wlc[...])
    wr0, wr1 = _halves(wr)
    o = lax.dot_general(sw0[...] * rw, wlw0, _CD, preferred_element_type=f32)
    o += lax.dot_general(sw1[...] * rw, wlw1, _CD, preferred_element_type=f32)
    o += lax.dot_general(sc0[...] * rc, wlc0, _CD, preferred_element_type=f32)
    o += lax.dot_general(sc1[...] * rc, wlc1, _CD, preferred_element_type=f32)
    o += lax.dot_general(xp0[...], wr0, _CD, preferred_element_type=f32)
    o += lax.dot_general(xp1[...], wr1, _CD, preferred_element_type=f32)
    o += bw[...] + bc[...]
    out[...] = jnp.where(o >= 0, o, 0.01 * o)


def _author_body(sr0, sr1, dr, xa0, xa1, wlr, wrr, br, out):
    rr = 1.0 / jnp.maximum(dr[...], 1.0)
    f32 = jnp.float32
    wlr0, wlr1 = _halves(wlr[...])
    wrr0, wrr1 = _halves(wrr[...])
    o = lax.dot_general(sr0[...] * rr, wlr0, _CD, preferred_element_type=f32)
    o += lax.dot_general(sr1[...] * rr, wlr1, _CD, preferred_element_type=f32)
    o += lax.dot_general(xa0[...], wrr0, _CD, preferred_element_type=f32)
    o += lax.dot_general(xa1[...], wrr1, _CD, preferred_element_type=f32)
    o += br[...]
    out[...] = jnp.where(o >= 0, o, 0.01 * o)


def _h0_spec():
    return pl.BlockSpec((R, HALF), lambda i, h: (i, 0))


def _h1_spec():
    return pl.BlockSpec((R, HALF), lambda i, h: (GRID + i, 0))


def _deg_spec():
    return pl.BlockSpec((R, 1), lambda i, h: (i, 0))


def _wslice_spec():
    return pl.BlockSpec((HALF, D), lambda i, h: (h, 0))


def _bslice_spec():
    return pl.BlockSpec((1, HALF), lambda i, h: (0, h))


def _fh_out_spec():
    return pl.BlockSpec((R, HALF), lambda i, h: (h * GRID + i, 0))


_FH_OUT = jax.ShapeDtypeStruct((2 * N, HALF), jnp.float32)

_paper_tc = pl.pallas_call(
    _paper_body,
    grid=(GRID, 2),
    in_specs=[
        _h0_spec(), _h1_spec(), _deg_spec(),
        _h0_spec(), _h1_spec(), _deg_spec(),
        _h0_spec(), _h1_spec(),
        _wslice_spec(), _wslice_spec(), _wslice_spec(), _wslice_spec(),
        _bslice_spec(), _bslice_spec(),
    ],
    out_specs=_fh_out_spec(),
    out_shape=_FH_OUT,
)

_author_tc = pl.pallas_call(
    _author_body,
    grid=(GRID, 2),
    in_specs=[
        _h0_spec(), _h1_spec(), _deg_spec(),
        _h0_spec(), _h1_spec(),
        _wslice_spec(), _wslice_spec(), _bslice_spec(),
    ],
    out_specs=_fh_out_spec(),
    out_shape=_FH_OUT,
)


def _pool_body(xa, xp, ba, bp, rep_a, rep_p):
    i = pl.program_id(1)
    iot = lax.broadcasted_iota(jnp.int32, (NG, R), 0)
    oh_a = (iot == ba[0]).astype(jnp.float32)
    oh_p = (iot == bp[0]).astype(jnp.float32)
    cd = (((1,), (0,)), ((), ()))
    pa = lax.dot_general(oh_a, xa[...], cd, preferred_element_type=jnp.float32)
    pp = lax.dot_general(oh_p, xp[...], cd, preferred_element_type=jnp.float32)

    @pl.when(i == 0)
    def _():
        rep_a[...] = pa
        rep_p[...] = pp

    @pl.when(i > 0)
    def _():
        rep_a[...] += pa
        rep_p[...] += pp


_pool_tc = pl.pallas_call(
    _pool_body,
    grid=(2, GRID),
    in_specs=[
        pl.BlockSpec((R, HALF), lambda h, i: (h * GRID + i, 0)),
        pl.BlockSpec((R, HALF), lambda h, i: (h * GRID + i, 0)),
        pl.BlockSpec((1, 1, R), lambda h, i: (i, 0, 0)),
        pl.BlockSpec((1, 1, R), lambda h, i: (i, 0, 0)),
    ],
    out_specs=[
        pl.BlockSpec((NG, HALF), lambda h, i: (0, h)),
        pl.BlockSpec((NG, HALF), lambda h, i: (0, h)),
    ],
    out_shape=[
        jax.ShapeDtypeStruct((NG, D), jnp.float32),
        jax.ShapeDtypeStruct((NG, D), jnp.float32),
    ],
)


def _head_body(rep_a, rep_p, wm, bm, wl, bl, out):
    f32 = jnp.float32
    wma = wm[:, 0:D]
    wmp = wm[:, D:2 * D]
    h = lax.dot_general(rep_a[...], wma, _CD, preferred_element_type=f32)
    h += lax.dot_general(rep_p[...], wmp, _CD, preferred_element_type=f32)
    h += bm[...]
    out[...] = lax.dot_general(h, wl[...], _CD,
                               preferred_element_type=f32) + bl[...]


_head_tc = pl.pallas_call(
    _head_body,
    out_shape=jax.ShapeDtypeStruct((NG, 128), jnp.float32),
)


def _flat_half(x):
    return jnp.concatenate([x[:, :HALF], x[:, HALF:]], axis=0)


def kernel(x_author, x_paper, edge_index_writes, edge_index_rev, edge_index_cites,
           batch_author, batch_paper,
           Wl0_writes, bl0_writes, Wr0_writes,
           Wl0_rev, bl0_rev, Wr0_rev,
           Wl0_cites, bl0_cites, Wr0_cites,
           Wl1_writes, bl1_writes, Wr1_writes,
           Wl1_rev, bl1_rev, Wr1_rev,
           Wl1_cites, bl1_cites, Wr1_cites,
           W_mlp, b_mlp, W_lin, b_lin):
    f32 = jnp.float32
    xa = x_author.astype(f32)
    xp = x_paper.astype(f32)
    ei_w = edge_index_writes.astype(jnp.int32)
    ei_r = edge_index_rev.astype(jnp.int32)
    ei_c = edge_index_cites.astype(jnp.int32)

    def comb(ei):
        s2 = ei[0].reshape(NT, NCHUNK, 1, C)
        d2 = ei[1].reshape(NT, NCHUNK, 1, C)
        return jnp.concatenate([s2, d2], axis=2).reshape(-1)

    edges = (comb(ei_w), comb(ei_c), comb(ei_r))

    xa_f = _flat_half(xa)
    xp_f = _flat_half(xp)
    sw_f, sc_f, sr_f, degs = _seg_l0(xa_f, xp_f, *edges)
    dw = degs[0, :, 0:1]
    dc = degs[1, :, 0:1]
    dr = degs[2, :, 0:1] + degs[3, :, 0:1]

    b = lambda v: v.reshape(1, -1).astype(f32)
    xp1_f = _paper_tc(sw_f, sw_f, dw, sc_f, sc_f, dc, xp_f, xp_f,
                      Wl0_writes, Wl0_cites, Wr0_writes, Wr0_cites,
                      b(bl0_writes), b(bl0_cites))
    xa1_f = _author_tc(sr_f, sr_f, dr, xa_f, xa_f,
                       Wl0_rev, Wr0_rev, b(bl0_rev))

    sw_f, sc_f, sr_f = _seg_l1(xa1_f, xp1_f, *edges)
    xp2_f = _paper_tc(sw_f, sw_f, dw, sc_f, sc_f, dc, xp1_f, xp1_f,
                      Wl1_writes, Wl1_cites, Wr1_writes, Wr1_cites,
                      b(bl1_writes), b(bl1_cites))
    xa2_f = _author_tc(sr_f, sr_f, dr, xa1_f, xa1_f,
                       Wl1_rev, Wr1_rev, b(bl1_rev))

    ba = batch_author.astype(jnp.int32).reshape(GRID, 1, R)
    bp = batch_paper.astype(jnp.int32).reshape(GRID, 1, R)
    rep_a, rep_p = _pool_tc(xa2_f, xp2_f, ba, bp)
    return _head_tc(rep_a, rep_p, W_mlp, b(b_mlp), W_lin, b(b_lin))


# speedup vs baseline: 1.0777x; 1.0777x over previous
import functools

import jax
import jax.numpy as jnp
from jax import lax
from jax.experimental import pallas as pl
from jax.experimental.pallas import tpu as pltpu
from jax.experimental.pallas import tpu_sc as plsc

N = 10000
E = 160000
D = 256
HALF = 128
NG = 64
NT = 16
EPT = E // NT
C = 80
NCHUNK = EPT // C
RCH = 80
NRC = N // RCH
RQ = (NRC + NT - 1) // NT


def _seg_body(with_deg, *refs):
    xa, xp, cw, cc, cr = refs[:5]
    rest = refs[5:]
    if with_deg:
        s_w, s_c, s_r, degs = rest[:4]
        rest = rest[4:]
    else:
        s_w, s_c, s_r = rest[:3]
        degs = None
        rest = rest[3:]
    acc = rest[0]
    cidx = list(rest[1:4])
    sidx = list(rest[4:7])
    didx = list(rest[7:10])
    rows = list(rest[10:13])
    semi, semg, sems = rest[13:16]

    c = lax.axis_index("c")
    t = lax.axis_index("s")
    xoff = c * N

    def fill_rows0(val):
        def _f(k, carry):
            rows[0][k // 8, pl.ds((k % 8) * 16, 16)] = jnp.full(
                (16,), val, jnp.float32)
            return carry
        lax.fori_loop(0, RCH * 8, _f, 0)

    def zero_acc():
        fill_rows0(0.0)
        for q in range(RQ):
            k = q * NT + t
            @pl.when(k < NRC)
            def _():
                pltpu.sync_copy(rows[0], acc.at[pl.ds(k * RCH, RCH)])

    def drain_acc(out_ref, slot, off):
        for q in range(RQ):
            k = q * NT + t
            @pl.when(k < NRC)
            def _():
                pltpu.sync_copy(acc.at[pl.ds(k * RCH, RCH)], rows[0])
                if slot is None:
                    pltpu.sync_copy(rows[0], out_ref.at[pl.ds(off + k * RCH, RCH)])
                else:
                    pltpu.sync_copy(rows[0], out_ref.at[slot, pl.ds(k * RCH, RCH)])

    def run_phase(comb_ref, out_ref, slot, x_ref, jlo=0, n=NCHUNK):
        gather = x_ref is not None
        zero_acc()
        if not gather:
            fill_rows0(1.0)
        plsc.subcore_barrier()
        tbase = t * (2 * EPT)

        def load_cidx(jj, p):
            pltpu.async_copy(
                comb_ref.at[pl.ds(tbase + (jlo + jj) * 2 * C, 2 * C)],
                cidx[p], semi)

        def build(jj, p):
            pltpu.make_async_copy(
                comb_ref.at[pl.ds(tbase + (jlo + jj) * 2 * C, 2 * C)],
                cidx[p], semi).wait()
            for k in range(C // 16):
                if gather:
                    sidx[p][pl.ds(k * 16, 16)] = (
                        cidx[p][pl.ds(k * 16, 16)] + xoff)
                didx[p][pl.ds(k * 16, 16)] = cidx[p][pl.ds(C + k * 16, 16)]

        def gath(p):
            pltpu.async_copy(x_ref.at[sidx[p]], rows[p], semg)

        def wait_gath(p):
            pltpu.make_async_copy(x_ref.at[sidx[p]], rows[p], semg).wait()

        def scat(p):
            src = rows[p] if gather else rows[0]
            pltpu.async_copy(src, acc.at[didx[p]], sems, add=True)

        def wait_scat(p):
            src = rows[p] if gather else rows[0]
            pltpu.make_async_copy(src, acc.at[didx[p]], sems).wait()

        load_cidx(0, 0)
        load_cidx(1, 1)
        build(0, 0)
        if gather:
            gath(0)

        def emit_iter(j, jj):
            p = j % 3
            p1 = (p + 1) % 3
            p2 = (p + 2) % 3
            if gather:
                wait_gath(p)
            scat(p)
            if j >= 2:
                wait_scat(p1)
            if j + 2 < n:
                load_cidx(jj + 2, p2)
            if j + 1 < n:
                build(jj + 1, p1)
                if gather:
                    gath(p1)

        steady = (n - 4) // 3
        tail = 3 * steady + 2
        emit_iter(0, 0)
        emit_iter(1, 1)

        def body(k, carry):
            j = 2 + 3 * k
            emit_iter(2, j)
            emit_iter(3, j + 1)
            emit_iter(4, j + 2)
            return carry
        lax.fori_loop(0, steady, body, 0)
        for j in range(tail, n):
            emit_iter(j, j)
        wait_scat((n - 2) % 3)
        wait_scat((n - 1) % 3)
        plsc.subcore_barrier()
        if slot is None:
            drain_acc(out_ref, None, xoff)
        else:
            drain_acc(out_ref, slot, 0)
        plsc.subcore_barrier()

    run_phase(cw, s_w, None, xa)
    run_phase(cc, s_c, None, xp)
    run_phase(cr, s_r, None, xp)
    if with_deg:
        @pl.when(c == 0)
        def _():
            run_phase(cw, degs, 0, None)
            run_phase(cr, degs, 2, None, 0, NCHUNK // 2)

        @pl.when(c == 1)
        def _():
            run_phase(cc, degs, 1, None)
            run_phase(cr, degs, 3, None, NCHUNK // 2, NCHUNK - NCHUNK // 2)


def _make_seg_kernel(with_deg):
    outs = [jax.ShapeDtypeStruct((2 * N, HALF), jnp.float32) for _ in range(3)]
    if with_deg:
        outs.append(jax.ShapeDtypeStruct((4, N, HALF), jnp.float32))
    scratch = (
        [pltpu.VMEM_SHARED((N, HALF), jnp.float32)]
        + [pltpu.VMEM((2 * C,), jnp.int32)] * 3
        + [pltpu.VMEM((C,), jnp.int32)] * 3
        + [pltpu.VMEM((C,), jnp.int32)] * 3
        + [pltpu.VMEM((C, HALF), jnp.float32)] * 3
        + [pltpu.SemaphoreType.DMA] * 3
    )
    mesh = plsc.VectorSubcoreMesh(core_axis_name="c", subcore_axis_name="s",
                                  num_cores=2, num_subcores=NT)
    return pl.kernel(
        functools.partial(_seg_body, with_deg),
        out_type=tuple(outs),
        mesh=mesh,
        scratch_types=scratch,
    )


_seg_l0 = _make_seg_kernel(True)
_seg_l1 = _make_seg_kernel(False)

R = 1000
GRID = N // R


_CD = (((1,), (1,)), ((), ()))


def _halves(x):
    return x[:, 0:HALF], x[:, HALF:D]


def _layer_body(sw0, sw1, dw, sc0, sc1, dc, sr0, sr1, dr, xp, xa,
                wlw, wlc, wrw, wrc, wlr, wrr, bwc, br, out_p, out_a):
    f32 = jnp.float32
    rw = 1.0 / jnp.maximum(dw[...], 1.0)
    rc = 1.0 / jnp.maximum(dc[...], 1.0)
    rr = 1.0 / jnp.maximum(dr[...], 1.0)
    mw = jnp.concatenate([sw0[...] * rw, sw1[...] * rw], axis=1)
    mc = jnp.concatenate([sc0[...] * rc, sc1[...] * rc], axis=1)
    mr = jnp.concatenate([sr0[...] * rr, sr1[...] * rr], axis=1)
    o = lax.dot_general(mw, wlw[...], _CD, preferred_element_type=f32)
    o += lax.dot_general(mc, wlc[...], _CD, preferred_element_type=f32)
    o += lax.dot_general(xp[...], wrw[...] + wrc[...], _CD,
                         preferred_element_type=f32)
    o += bwc[...]
    out_p[...] = jnp.where(o >= 0, o, 0.01 * o)
    o = lax.dot_general(mr, wlr[...], _CD, preferred_element_type=f32)
    o += lax.dot_general(xa[...], wrr[...], _CD, preferred_element_type=f32)
    o += br[...]
    out_a[...] = jnp.where(o >= 0, o, 0.01 * o)


def _h0_spec():
    return pl.BlockSpec((R, HALF), lambda i: (i, 0))


def _h1_spec():
    return pl.BlockSpec((R, HALF), lambda i: (GRID + i, 0))


def _deg_spec():
    return pl.BlockSpec((R, 1), lambda i: (i, 0))


def _row_spec():
    return pl.BlockSpec((R, D), lambda i: (i, 0))


def _full_spec(shape):
    nd = len(shape)
    return pl.BlockSpec(shape, lambda i: (0,) * nd)


_layer_tc = pl.pallas_call(
    _layer_body,
    grid=(GRID,),
    in_specs=[
        _h0_spec(), _h1_spec(), _deg_spec(),
        _h0_spec(), _h1_spec(), _deg_spec(),
        _h0_spec(), _h1_spec(), _deg_spec(),
        _row_spec(), _row_spec(),
        _full_spec((D, D)), _full_spec((D, D)), _full_spec((D, D)),
        _full_spec((D, D)), _full_spec((D, D)), _full_spec((D, D)),
        _full_spec((1, D)), _full_spec((1, D)),
    ],
    out_specs=[_row_spec(), _row_spec()],
    out_shape=[jax.ShapeDtypeStruct((N, D), jnp.float32),
               jax.ShapeDtypeStruct((N, D), jnp.float32)],
)


def _poolhead_body(xa, xp, ba, bp, wm, bm, wl, bl, out, rep):
    i = pl.program_id(0)
    f32 = jnp.float32
    iot = lax.broadcasted_iota(jnp.int32, (NG, R), 0)
    oh_a = (iot == ba[0]).astype(f32)
    oh_p = (iot == bp[0]).astype(f32)
    cd = (((1,), (0,)), ((), ()))
    pa = lax.dot_general(oh_a, xa[...], cd, preferred_element_type=f32)
    pp = lax.dot_general(oh_p, xp[...], cd, preferred_element_type=f32)

    @pl.when(i == 0)
    def _():
        rep[:, 0:D] = pa
        rep[:, D:2 * D] = pp

    @pl.when(i > 0)
    def _():
        rep[:, 0:D] += pa
        rep[:, D:2 * D] += pp

    @pl.when(i == GRID - 1)
    def _():
        h = lax.dot_general(rep[...], wm[...], _CD,
                            preferred_element_type=f32) + bm[...]
        out[...] = lax.dot_general(h, wl[...], _CD,
                                   preferred_element_type=f32) + bl[...]


_poolhead_tc = pl.pallas_call(
    _poolhead_body,
    grid=(GRID,),
    in_specs=[
        _row_spec(), _row_spec(),
        pl.BlockSpec((1, 1, R), lambda i: (i, 0, 0)),
        pl.BlockSpec((1, 1, R), lambda i: (i, 0, 0)),
        _full_spec((D, 2 * D)), _full_spec((1, D)),
        _full_spec((128, D)), _full_spec((1, 128)),
    ],
    out_specs=_full_spec((NG, 128)),
    out_shape=jax.ShapeDtypeStruct((NG, 128), jnp.float32),
    scratch_shapes=[pltpu.VMEM((NG, 2 * D), jnp.float32)],
)


def _flat_half(x):
    return jnp.concatenate([x[:, :HALF], x[:, HALF:]], axis=0)


def kernel(x_author, x_paper, edge_index_writes, edge_index_rev, edge_index_cites,
           batch_author, batch_paper,
           Wl0_writes, bl0_writes, Wr0_writes,
           Wl0_rev, bl0_rev, Wr0_rev,
           Wl0_cites, bl0_cites, Wr0_cites,
           Wl1_writes, bl1_writes, Wr1_writes,
           Wl1_rev, bl1_rev, Wr1_rev,
           Wl1_cites, bl1_cites, Wr1_cites,
           W_mlp, b_mlp, W_lin, b_lin):
    f32 = jnp.float32
    xa = x_author.astype(f32)
    xp = x_paper.astype(f32)
    ei_w = edge_index_writes.astype(jnp.int32)
    ei_r = edge_index_rev.astype(jnp.int32)
    ei_c = edge_index_cites.astype(jnp.int32)

    def comb(ei):
        s2 = ei[0].reshape(NT, NCHUNK, 1, C)
        d2 = ei[1].reshape(NT, NCHUNK, 1, C)
        return jnp.concatenate([s2, d2], axis=2).reshape(-1)

    edges = (comb(ei_w), comb(ei_c), comb(ei_r))

    xa_f = _flat_half(xa)
    xp_f = _flat_half(xp)
    sw_f, sc_f, sr_f, degs = _seg_l0(xa_f, xp_f, *edges)
    dw = degs[0, :, 0:1]
    dc = degs[1, :, 0:1]
    dr = degs[2, :, 0:1] + degs[3, :, 0:1]

    b = lambda v: v.reshape(1, -1).astype(f32)
    xp1, xa1 = _layer_tc(sw_f, sw_f, dw, sc_f, sc_f, dc, sr_f, sr_f, dr,
                         xp, xa,
                         Wl0_writes, Wl0_cites, Wr0_writes, Wr0_cites,
                         Wl0_rev, Wr0_rev,
                         b(bl0_writes + bl0_cites), b(bl0_rev))

    sw_f, sc_f, sr_f = _seg_l1(_flat_half(xa1), _flat_half(xp1), *edges)
    xp2, xa2 = _layer_tc(sw_f, sw_f, dw, sc_f, sc_f, dc, sr_f, sr_f, dr,
                         xp1, xa1,
                         Wl1_writes, Wl1_cites, Wr1_writes, Wr1_cites,
                         Wl1_rev, Wr1_rev,
                         b(bl1_writes + bl1_cites), b(bl1_rev))

    ba = batch_author.astype(jnp.int32).reshape(GRID, 1, R)
    bp = batch_paper.astype(jnp.int32).reshape(GRID, 1, R)
    return _poolhead_tc(xa2, xp2, ba, bp, W_mlp, b(b_mlp), W_lin, b(b_lin))


# speedup vs baseline: 1.0847x; 1.0064x over previous
import functools

import jax
import jax.numpy as jnp
from jax import lax
from jax.experimental import pallas as pl
from jax.experimental.pallas import tpu as pltpu
from jax.experimental.pallas import tpu_sc as plsc

N = 10000
E = 160000
D = 256
HALF = 128
NG = 64
NT = 16
EPT = E // NT
C = 80
NCHUNK = EPT // C
RCH = 80
NRC = N // RCH
RQ = (NRC + NT - 1) // NT


def _seg_body(with_deg, *refs):
    xa, xp, cw, cc, cr = refs[:5]
    rest = refs[5:]
    if with_deg:
        s_w, s_c, s_r, degs = rest[:4]
        rest = rest[4:]
    else:
        s_w, s_c, s_r = rest[:3]
        degs = None
        rest = rest[3:]
    acc = rest[0]
    cidx = list(rest[1:4])
    sidx = list(rest[4:7])
    didx = list(rest[7:10])
    rows = list(rest[10:13])
    semi, semg, sems = rest[13:16]

    c = lax.axis_index("c")
    t = lax.axis_index("s")
    xoff = c * N

    def fill_rows0(val):
        def _f(k, carry):
            rows[0][k // 8, pl.ds((k % 8) * 16, 16)] = jnp.full(
                (16,), val, jnp.float32)
            return carry
        lax.fori_loop(0, RCH * 8, _f, 0)

    def zero_acc():
        fill_rows0(0.0)
        for q in range(RQ):
            k = q * NT + t
            @pl.when(k < NRC)
            def _():
                pltpu.sync_copy(rows[0], acc.at[pl.ds(k * RCH, RCH)])

    def drain_acc(out_ref, slot, off):
        for q in range(RQ):
            k = q * NT + t
            @pl.when(k < NRC)
            def _():
                if slot is None:
                    pltpu.sync_copy(acc.at[pl.ds(k * RCH, RCH)],
                                    out_ref.at[pl.ds(off + k * RCH, RCH)])
                else:
                    pltpu.sync_copy(acc.at[pl.ds(k * RCH, RCH)],
                                    out_ref.at[slot, pl.ds(k * RCH, RCH)])

    def run_phase(comb_ref, out_ref, slot, x_ref, jlo=0, n=NCHUNK):
        gather = x_ref is not None
        zero_acc()
        if not gather:
            fill_rows0(1.0)
        plsc.subcore_barrier()
        tbase = t * (2 * EPT)

        def load_cidx(jj, p):
            pltpu.async_copy(
                comb_ref.at[pl.ds(tbase + (jlo + jj) * 2 * C, 2 * C)],
                cidx[p], semi)

        def build(jj, p):
            pltpu.make_async_copy(
                comb_ref.at[pl.ds(tbase + (jlo + jj) * 2 * C, 2 * C)],
                cidx[p], semi).wait()
            for k in range(C // 16):
                if gather:
                    sidx[p][pl.ds(k * 16, 16)] = (
                        cidx[p][pl.ds(k * 16, 16)] + xoff)
                didx[p][pl.ds(k * 16, 16)] = cidx[p][pl.ds(C + k * 16, 16)]

        def gath(p):
            pltpu.async_copy(x_ref.at[sidx[p]], rows[p], semg)

        def wait_gath(p):
            pltpu.make_async_copy(x_ref.at[sidx[p]], rows[p], semg).wait()

        def scat(p):
            src = rows[p] if gather else rows[0]
            pltpu.async_copy(src, acc.at[didx[p]], sems, add=True)

        def wait_scat(p):
            src = rows[p] if gather else rows[0]
            pltpu.make_async_copy(src, acc.at[didx[p]], sems).wait()

        load_cidx(0, 0)
        load_cidx(1, 1)
        build(0, 0)
        if gather:
            gath(0)

        def emit_iter(j, jj):
            p = j % 3
            p1 = (p + 1) % 3
            p2 = (p + 2) % 3
            if gather:
                wait_gath(p)
            scat(p)
            if j >= 2:
                wait_scat(p1)
            if j + 2 < n:
                load_cidx(jj + 2, p2)
            if j + 1 < n:
                build(jj + 1, p1)
                if gather:
                    gath(p1)

        steady = (n - 4) // 3
        tail = 3 * steady + 2
        emit_iter(0, 0)
        emit_iter(1, 1)

        def body(k, carry):
            j = 2 + 3 * k
            emit_iter(2, j)
            emit_iter(3, j + 1)
            emit_iter(4, j + 2)
            return carry
        lax.fori_loop(0, steady, body, 0)
        for j in range(tail, n):
            emit_iter(j, j)
        wait_scat((n - 2) % 3)
        wait_scat((n - 1) % 3)
        plsc.subcore_barrier()
        if slot is None:
            drain_acc(out_ref, None, xoff)
        else:
            drain_acc(out_ref, slot, 0)

    run_phase(cw, s_w, None, xa)
    run_phase(cc, s_c, None, xp)
    run_phase(cr, s_r, None, xp)
    if with_deg:
        @pl.when(c == 0)
        def _():
            run_phase(cw, degs, 0, None)
            run_phase(cr, degs, 2, None, 0, NCHUNK // 2)

        @pl.when(c == 1)
        def _():
            run_phase(cc, degs, 1, None)
            run_phase(cr, degs, 3, None, NCHUNK // 2, NCHUNK - NCHUNK // 2)


def _make_seg_kernel(with_deg):
    outs = [jax.ShapeDtypeStruct((2 * N, HALF), jnp.float32) for _ in range(3)]
    if with_deg:
        outs.append(jax.ShapeDtypeStruct((4, N, HALF), jnp.float32))
    scratch = (
        [pltpu.VMEM_SHARED((N, HALF), jnp.float32)]
        + [pltpu.VMEM((2 * C,), jnp.int32)] * 3
        + [pltpu.VMEM((C,), jnp.int32)] * 3
        + [pltpu.VMEM((C,), jnp.int32)] * 3
        + [pltpu.VMEM((C, HALF), jnp.float32)] * 3
        + [pltpu.SemaphoreType.DMA] * 3
    )
    mesh = plsc.VectorSubcoreMesh(core_axis_name="c", subcore_axis_name="s",
                                  num_cores=2, num_subcores=NT)
    return pl.kernel(
        functools.partial(_seg_body, with_deg),
        out_type=tuple(outs),
        mesh=mesh,
        scratch_types=scratch,
    )


_seg_l0 = _make_seg_kernel(True)
_seg_l1 = _make_seg_kernel(False)

R = 1000
GRID = N // R


_CD = (((1,), (1,)), ((), ()))


def _halves(x):
    return x[:, 0:HALF], x[:, HALF:D]


def _layer_body(sw0, sw1, dw, sc0, sc1, dc, sr0, sr1, dr, xp, xa,
                wlw, wlc, wrw, wrc, wlr, wrr, bwc, br, out_p, out_a):
    f32 = jnp.float32
    rw = 1.0 / jnp.maximum(dw[...], 1.0)
    rc = 1.0 / jnp.maximum(dc[...], 1.0)
    rr = 1.0 / jnp.maximum(dr[...], 1.0)
    mw = jnp.concatenate([sw0[...] * rw, sw1[...] * rw], axis=1)
    mc = jnp.concatenate([sc0[...] * rc, sc1[...] * rc], axis=1)
    mr = jnp.concatenate([sr0[...] * rr, sr1[...] * rr], axis=1)
    o = lax.dot_general(mw, wlw[...], _CD, preferred_element_type=f32)
    o += lax.dot_general(mc, wlc[...], _CD, preferred_element_type=f32)
    o += lax.dot_general(xp[...], wrw[...] + wrc[...], _CD,
                         preferred_element_type=f32)
    o += bwc[...]
    out_p[...] = jnp.where(o >= 0, o, 0.01 * o)
    o = lax.dot_general(mr, wlr[...], _CD, preferred_element_type=f32)
    o += lax.dot_general(xa[...], wrr[...], _CD, preferred_element_type=f32)
    o += br[...]
    out_a[...] = jnp.where(o >= 0, o, 0.01 * o)


def _h0_spec():
    return pl.BlockSpec((R, HALF), lambda i: (i, 0))


def _h1_spec():
    return pl.BlockSpec((R, HALF), lambda i: (GRID + i, 0))


def _deg_spec():
    return pl.BlockSpec((R, 1), lambda i: (i, 0))


def _row_spec():
    return pl.BlockSpec((R, D), lambda i: (i, 0))


def _full_spec(shape):
    nd = len(shape)
    return pl.BlockSpec(shape, lambda i: (0,) * nd)


_layer_tc = pl.pallas_call(
    _layer_body,
    grid=(GRID,),
    in_specs=[
        _h0_spec(), _h1_spec(), _deg_spec(),
        _h0_spec(), _h1_spec(), _deg_spec(),
        _h0_spec(), _h1_spec(), _deg_spec(),
        _row_spec(), _row_spec(),
        _full_spec((D, D)), _full_spec((D, D)), _full_spec((D, D)),
        _full_spec((D, D)), _full_spec((D, D)), _full_spec((D, D)),
        _full_spec((1, D)), _full_spec((1, D)),
    ],
    out_specs=[_row_spec(), _row_spec()],
    out_shape=[jax.ShapeDtypeStruct((N, D), jnp.float32),
               jax.ShapeDtypeStruct((N, D), jnp.float32)],
)


def _poolhead_body(xa, xp, ba, bp, wm, bm, wl, bl, out, rep):
    i = pl.program_id(0)
    f32 = jnp.float32
    iot = lax.broadcasted_iota(jnp.int32, (NG, R), 0)
    oh_a = (iot == ba[0]).astype(f32)
    oh_p = (iot == bp[0]).astype(f32)
    cd = (((1,), (0,)), ((), ()))
    pa = lax.dot_general(oh_a, xa[...], cd, preferred_element_type=f32)
    pp = lax.dot_general(oh_p, xp[...], cd, preferred_element_type=f32)

    @pl.when(i == 0)
    def _():
        rep[:, 0:D] = pa
        rep[:, D:2 * D] = pp

    @pl.when(i > 0)
    def _():
        rep[:, 0:D] += pa
        rep[:, D:2 * D] += pp

    @pl.when(i == GRID - 1)
    def _():
        h = lax.dot_general(rep[...], wm[...], _CD,
                            preferred_element_type=f32) + bm[...]
        out[...] = lax.dot_general(h, wl[...], _CD,
                                   preferred_element_type=f32) + bl[...]


_poolhead_tc = pl.pallas_call(
    _poolhead_body,
    grid=(GRID,),
    in_specs=[
        _row_spec(), _row_spec(),
        pl.BlockSpec((1, 1, R), lambda i: (i, 0, 0)),
        pl.BlockSpec((1, 1, R), lambda i: (i, 0, 0)),
        _full_spec((D, 2 * D)), _full_spec((1, D)),
        _full_spec((128, D)), _full_spec((1, 128)),
    ],
    out_specs=_full_spec((NG, 128)),
    out_shape=jax.ShapeDtypeStruct((NG, 128), jnp.float32),
    scratch_shapes=[pltpu.VMEM((NG, 2 * D), jnp.float32)],
)


def _flat_half(x):
    return jnp.concatenate([x[:, :HALF], x[:, HALF:]], axis=0)


def kernel(x_author, x_paper, edge_index_writes, edge_index_rev, edge_index_cites,
           batch_author, batch_paper,
           Wl0_writes, bl0_writes, Wr0_writes,
           Wl0_rev, bl0_rev, Wr0_rev,
           Wl0_cites, bl0_cites, Wr0_cites,
           Wl1_writes, bl1_writes, Wr1_writes,
           Wl1_rev, bl1_rev, Wr1_rev,
           Wl1_cites, bl1_cites, Wr1_cites,
           W_mlp, b_mlp, W_lin, b_lin):
    f32 = jnp.float32
    xa = x_author.astype(f32)
    xp = x_paper.astype(f32)
    ei_w = edge_index_writes.astype(jnp.int32)
    ei_r = edge_index_rev.astype(jnp.int32)
    ei_c = edge_index_cites.astype(jnp.int32)

    def comb(ei):
        s2 = ei[0].reshape(NT, NCHUNK, 1, C)
        d2 = ei[1].reshape(NT, NCHUNK, 1, C)
        return jnp.concatenate([s2, d2], axis=2).reshape(-1)

    edges = (comb(ei_w), comb(ei_c), comb(ei_r))

    xa_f = _flat_half(xa)
    xp_f = _flat_half(xp)
    sw_f, sc_f, sr_f, degs = _seg_l0(xa_f, xp_f, *edges)
    dw = degs[0, :, 0:1]
    dc = degs[1, :, 0:1]
    dr = degs[2, :, 0:1] + degs[3, :, 0:1]

    b = lambda v: v.reshape(1, -1).astype(f32)
    xp1, xa1 = _layer_tc(sw_f, sw_f, dw, sc_f, sc_f, dc, sr_f, sr_f, dr,
                         xp, xa,
                         Wl0_writes, Wl0_cites, Wr0_writes, Wr0_cites,
                         Wl0_rev, Wr0_rev,
                         b(bl0_writes + bl0_cites), b(bl0_rev))

    sw_f, sc_f, sr_f = _seg_l1(_flat_half(xa1), _flat_half(xp1), *edges)
    xp2, xa2 = _layer_tc(sw_f, sw_f, dw, sc_f, sc_f, dc, sr_f, sr_f, dr,
                         xp1, xa1,
                         Wl1_writes, Wl1_cites, Wr1_writes, Wr1_cites,
                         Wl1_rev, Wr1_rev,
                         b(bl1_writes + bl1_cites), b(bl1_rev))

    ba = batch_author.astype(jnp.int32).reshape(GRID, 1, R)
    bp = batch_paper.astype(jnp.int32).reshape(GRID, 1, R)
    return _poolhead_tc(xa2, xp2, ba, bp, W_mlp, b(b_mlp), W_lin, b(b_lin))
